# Initial kernel scaffold; baseline (speedup 1.0000x reference)
import jax
import jax.numpy as jnp
from jax.experimental import pallas as pl


def kernel(feature, edge_index, adj_values, weight):
    raise NotImplementedError("write your pallas kernel here")


# speedup vs baseline: 3.6366x; 3.6366x over previous
import functools

import jax
import jax.numpy as jnp
from jax import lax
from jax.experimental import pallas as pl
from jax.experimental.pallas import tpu as pltpu
from jax.experimental.pallas import tpu_sc as plsc

N = 10000
E = 320000
D = 128
NC = 2
NS = 16
NW = NC * NS
EPW = E // NW
CHUNK = 128
NCHUNK = 80
EPW_PAD = NCHUNK * CHUNK
SUP = 16
NSUP = NCHUNK // SUP
CROWS = NW * NCHUNK
N_PAD = 10240
RPT = N_PAD // NS
ZROWS = 128

_mesh = plsc.VectorSubcoreMesh(core_axis_name="c", subcore_axis_name="s")


@functools.partial(
    pl.kernel,
    out_type=jax.ShapeDtypeStruct((NC, N_PAD, D), jnp.float32),
    mesh=_mesh,
    scratch_types=[
        pltpu.VMEM((SUP, CHUNK), jnp.int32),
        pltpu.VMEM((SUP, CHUNK), jnp.int32),
        pltpu.VMEM((SUP, CHUNK), jnp.float32),
        pltpu.VMEM((CHUNK, D), jnp.float32),
        pltpu.VMEM_SHARED((N_PAD, D), jnp.float32),
        pltpu.SemaphoreType.DMA,
    ],
)
def _sc_agg(src_hbm, dst_hbm, adj_hbm, feat_hbm, out_hbm,
            src_v, dst_v, adj_v, rows_v, agg_sh, sem):
    c = lax.axis_index("c")
    s = lax.axis_index("s")
    wid = c * NS + s
    iota16 = lax.broadcasted_iota(jnp.int32, (16,), 0)

    zero = jnp.zeros((16,), jnp.float32)

    def zrow(r, carry):
        for cc in range(8):
            rows_v[r, pl.ds(cc * 16, 16)] = zero
        return carry

    lax.fori_loop(0, ZROWS, zrow, 0)
    base = s * RPT
    for q in range(RPT // ZROWS):
        pltpu.sync_copy(rows_v, agg_sh.at[pl.ds(base + q * ZROWS, ZROWS)])
    plsc.subcore_barrier()

    def superblk(sb, carry):
        rvec = iota16 + (wid * NCHUNK + sb * SUP)
        pltpu.async_copy(src_hbm.at[rvec], src_v, sem).wait()
        pltpu.async_copy(dst_hbm.at[rvec], dst_v, sem).wait()
        pltpu.async_copy(adj_hbm.at[rvec], adj_v, sem).wait()

        def chunk(j, cr):
            pltpu.async_copy(feat_hbm.at[src_v.at[j]], rows_v, sem).wait()

            def sgroup(g, cr2):
                av = adj_v[j, pl.ds(g * 16, 16)]
                rbase = g * 16
                for rr in range(16):
                    a = av[rr]
                    r = rbase + rr
                    for cc in range(8):
                        sl = pl.ds(cc * 16, 16)
                        rows_v[r, sl] = rows_v[r, sl] * a
                return cr2

            lax.fori_loop(0, CHUNK // 16, sgroup, 0)

            pltpu.sync_copy(rows_v, agg_sh.at[dst_v.at[j]], add=True)
            return cr

        lax.fori_loop(0, SUP, chunk, 0)
        return carry

    lax.fori_loop(0, NSUP, superblk, 0)
    plsc.subcore_barrier()

    for q in range(RPT // ZROWS):
        off = base + q * ZROWS
        pltpu.sync_copy(agg_sh.at[pl.ds(off, ZROWS)], rows_v)
        pltpu.sync_copy(rows_v, out_hbm.at[c].at[pl.ds(off, ZROWS)])


BLK = 1000


def _tc_body(agg_ref, feat_ref, w_ref, o_ref):
    a = (agg_ref[0] + agg_ref[1] + feat_ref[...]) * 0.5
    o_ref[...] = jnp.dot(a, w_ref[...], preferred_element_type=jnp.float32)


def _tc_finish(agg, feature, weight):
    return pl.pallas_call(
        _tc_body,
        out_shape=jax.ShapeDtypeStruct((N, D), jnp.float32),
        grid=(N // BLK,),
        in_specs=[
            pl.BlockSpec((NC, BLK, D), lambda i: (0, i, 0)),
            pl.BlockSpec((BLK, D), lambda i: (i, 0)),
            pl.BlockSpec((D, D), lambda i: (0, 0)),
        ],
        out_specs=pl.BlockSpec((BLK, D), lambda i: (i, 0)),
    )(agg, feature, weight)


def _to_rows(x):
    x = x.reshape(NW, EPW)
    x = jnp.pad(x, ((0, 0), (0, EPW_PAD - EPW)))
    return x.reshape(-1, CHUNK)


def kernel(feature, edge_index, adj_values, weight):
    src = _to_rows(edge_index[1].astype(jnp.int32))
    dst = _to_rows(edge_index[0].astype(jnp.int32))
    adj = _to_rows(adj_values)
    agg = _sc_agg(src, dst, adj, feature)
    return _tc_finish(agg, feature, weight)


# speedup vs baseline: 4.2489x; 1.1684x over previous
import functools

import jax
import jax.numpy as jnp
from jax import lax
from jax.experimental import pallas as pl
from jax.experimental.pallas import tpu as pltpu
from jax.experimental.pallas import tpu_sc as plsc

N = 10000
E = 320000
D = 128
NC = 2
NS = 16
NW = NC * NS
EPW = E // NW
CHUNK = 128
NCHUNK = 80
EPW_PAD = NCHUNK * CHUNK
SUP = 16
NSUP = NCHUNK // SUP
N_PAD = 10240
RPT = N_PAD // NS
ZROWS = 128

_mesh = plsc.VectorSubcoreMesh(core_axis_name="c", subcore_axis_name="s")


@functools.partial(
    pl.kernel,
    out_type=jax.ShapeDtypeStruct((NC, N_PAD, D), jnp.float32),
    mesh=_mesh,
    scratch_types=[
        pltpu.VMEM((SUP, CHUNK), jnp.int32),
        pltpu.VMEM((SUP, CHUNK), jnp.int32),
        pltpu.VMEM((SUP, CHUNK), jnp.float32),
        pltpu.VMEM((2, CHUNK, D), jnp.float32),
        pltpu.VMEM_SHARED((N_PAD, D), jnp.float32),
        pltpu.SemaphoreType.DMA,
        pltpu.SemaphoreType.DMA,
        pltpu.SemaphoreType.DMA,
        pltpu.SemaphoreType.DMA,
    ],
)
def _sc_agg(src_hbm, dst_hbm, adj_hbm, feat_hbm, out_hbm,
            src_v, dst_v, adj_v, rows_v, agg_sh,
            gsem0, gsem1, ssem0, ssem1):
    c = lax.axis_index("c")
    s = lax.axis_index("s")
    wid = c * NS + s
    gsem = (gsem0, gsem1)
    ssem = (ssem0, ssem1)

    zero = jnp.zeros((16,), jnp.float32)

    def zrow(r, carry):
        for cc in range(8):
            rows_v[0, r, pl.ds(cc * 16, 16)] = zero
        return carry

    lax.fori_loop(0, ZROWS, zrow, 0)
    base = s * RPT
    for q in range(RPT // ZROWS):
        pltpu.sync_copy(rows_v.at[0], agg_sh.at[pl.ds(base + q * ZROWS, ZROWS)])
    plsc.subcore_barrier()

    def g_fire(jj, b):
        pltpu.async_copy(feat_hbm.at[src_v.at[jj]], rows_v.at[b], gsem[b])

    def g_wait(jj, b):
        pltpu.make_async_copy(
            feat_hbm.at[src_v.at[jj]], rows_v.at[b], gsem[b]).wait()

    def s_fire(jj, b):
        pltpu.async_copy(rows_v.at[b], agg_sh.at[dst_v.at[jj]],
                         ssem[b], add=True)

    def s_wait(jj, b):
        pltpu.make_async_copy(
            rows_v.at[b], agg_sh.at[dst_v.at[jj]], ssem[b]).wait()

    def scale(jj, b):
        def sgroup(g, cr):
            av = adj_v[jj, pl.ds(g * 16, 16)]
            rbase = g * 16
            for rr in range(16):
                a = av[rr]
                r = rbase + rr
                for cc in range(8):
                    sl = pl.ds(cc * 16, 16)
                    rows_v[b, r, sl] = rows_v[b, r, sl] * a
            return cr

        lax.fori_loop(0, CHUNK // 16, sgroup, 0)

    def chunk_body(jj, b, fire_next):
        g_wait(jj, b)
        scale(jj, b)
        s_fire(jj, b)
        s_wait(jj, b)
        if fire_next:
            g_fire(jj + 2, b)

    def superblk(sb, carry):
        e0 = sb * SUP
        pltpu.sync_copy(src_hbm.at[wid, pl.ds(e0, SUP)], src_v)
        pltpu.sync_copy(dst_hbm.at[wid, pl.ds(e0, SUP)], dst_v)
        pltpu.sync_copy(adj_hbm.at[wid, pl.ds(e0, SUP)], adj_v)
        g_fire(0, 0)
        g_fire(1, 1)

        def pair(p, cr):
            j0 = p * 2
            chunk_body(j0, 0, True)
            chunk_body(j0 + 1, 1, True)
            return cr

        lax.fori_loop(0, SUP // 2 - 1, pair, 0)
        chunk_body(SUP - 2, 0, False)
        chunk_body(SUP - 1, 1, False)
        return carry

    lax.fori_loop(0, NSUP, superblk, 0)
    plsc.subcore_barrier()

    for q in range(RPT // ZROWS):
        off = base + q * ZROWS
        pltpu.sync_copy(agg_sh.at[pl.ds(off, ZROWS)], rows_v.at[0])
        pltpu.sync_copy(rows_v.at[0], out_hbm.at[c].at[pl.ds(off, ZROWS)])


BLK = 1000


def _tc_body(agg_ref, feat_ref, w_ref, o_ref):
    a = (agg_ref[0] + agg_ref[1] + feat_ref[...]) * 0.5
    o_ref[...] = jnp.dot(a, w_ref[...], preferred_element_type=jnp.float32)


def _tc_finish(agg, feature, weight):
    return pl.pallas_call(
        _tc_body,
        out_shape=jax.ShapeDtypeStruct((N, D), jnp.float32),
        grid=(N // BLK,),
        in_specs=[
            pl.BlockSpec((NC, BLK, D), lambda i: (0, i, 0)),
            pl.BlockSpec((BLK, D), lambda i: (i, 0)),
            pl.BlockSpec((D, D), lambda i: (0, 0)),
        ],
        out_specs=pl.BlockSpec((BLK, D), lambda i: (i, 0)),
    )(agg, feature, weight)


def _to_chunks(x):
    x = x.reshape(NW, EPW)
    x = jnp.pad(x, ((0, 0), (0, EPW_PAD - EPW)))
    return x.reshape(NW, NCHUNK, CHUNK)


def kernel(feature, edge_index, adj_values, weight):
    src = _to_chunks(edge_index[1].astype(jnp.int32))
    dst = _to_chunks(edge_index[0].astype(jnp.int32))
    adj = _to_chunks(adj_values)
    agg = _sc_agg(src, dst, adj, feature)
    return _tc_finish(agg, feature, weight)
